# Initial kernel scaffold; baseline (speedup 1.0000x reference)
import jax
import jax.numpy as jnp
from jax.experimental import pallas as pl


def kernel(x, p):
    raise NotImplementedError("write your pallas kernel here")


# speedup vs baseline: 20.9536x; 20.9536x over previous
import functools

import jax
import jax.numpy as jnp
from jax import lax
from jax.experimental import pallas as pl
from jax.experimental.pallas import tpu as pltpu
from jax.experimental.pallas import tpu_sc as plsc

NC = 2
NS = 16
NW = NC * NS
L = 16

N_TOTAL = 4096 * 4096
E_PER_W = N_TOTAL // NW
CHUNK = 8192
NCHUNKS = E_PER_W // CHUNK

PASS_CFG = (
    (20, 4096, None, 1),
    (10, 1024, 20, 4),
    (0, 1024, 10, 4),
)

def _biased_key(v):
  b = plsc.bitcast(v, jnp.int32)
  int_min = jnp.int32(-2147483648)
  neg = int_min - b
  key = jnp.where(b < 0, neg, b)
  return lax.bitwise_xor(key, int_min)


def _make_hist_pass(mesh, digit_shift, nbins, prefix_shift, nsearch):
  scratch = [
      pltpu.VMEM((CHUNK,), jnp.float32),
      pltpu.VMEM((nsearch * L,), jnp.int32),
      pltpu.VMEM((nsearch * nbins,), jnp.int32),
  ]
  for _ in range(nsearch):
    scratch.append(pltpu.VMEM((nbins * L,), jnp.int32))

  @functools.partial(
      pl.kernel,
      out_type=jax.ShapeDtypeStruct((NW, nsearch * nbins), jnp.int32),
      mesh=mesh,
      scratch_types=scratch,
      compiler_params=pltpu.CompilerParams(needs_layout_passes=False),
  )
  def hist_pass(x_hbm, params_hbm, out_hbm, buf, pbuf, outbuf, *hists):
    wid = lax.axis_index("s") * NC + lax.axis_index("c")
    lane = lax.iota(jnp.int32, L)
    ones = jnp.ones((L,), jnp.int32)
    zeros16 = jnp.zeros((L,), jnp.int32)

    def zero_body(i, carry):
      for h in hists:
        h[pl.ds(i * L, L)] = zeros16
      return carry

    lax.fori_loop(0, nbins, zero_body, 0)

    pltpu.sync_copy(params_hbm, pbuf)
    pvs = [pbuf[pl.ds(s * L, L)] for s in range(nsearch)]

    base = wid * E_PER_W

    def chunk_body(c, carry):
      pltpu.sync_copy(x_hbm.at[pl.ds(base + c * CHUNK, CHUNK)], buf)

      def vec_body(i, inner):
        v = buf[pl.ds(i * L, L)]
        ku = _biased_key(v)
        dig = lax.shift_right_logical(ku, digit_shift)
        if prefix_shift is None:
          idx = dig * L + lane
          plsc.addupdate_scatter(hists[0], [idx], ones)
        else:
          dig = lax.bitwise_and(dig, jnp.int32(nbins - 1))
          idx = dig * L + lane
          pfx = lax.shift_right_logical(ku, prefix_shift)
          for s in range(nsearch):
            plsc.addupdate_scatter(hists[s], [idx], ones, mask=pfx == pvs[s])
        return inner

      lax.fori_loop(0, CHUNK // L, vec_body, 0)
      return carry

    lax.fori_loop(0, NCHUNKS, chunk_body, 0)

    for s in range(nsearch):

      def red_body(g, carry, s=s):
        bin_base = (g * L + lane) * L
        acc = jnp.zeros((L,), jnp.int32)
        for l in range(L):
          acc = acc + plsc.load_gather(hists[s], [bin_base + l])
        outbuf[pl.ds(s * nbins + g * L, L)] = acc
        return carry

      lax.fori_loop(0, nbins // L, red_body, 0)

    pltpu.sync_copy(outbuf, out_hbm.at[wid])

  return hist_pass


@functools.cache
def _get_hist_passes():
  mesh = plsc.VectorSubcoreMesh(
      core_axis_name="c", subcore_axis_name="s", num_cores=NC, num_subcores=NS
  )
  return tuple(_make_hist_pass(mesh, *cfg) for cfg in PASS_CFG)


def _searchsorted_rows(cum, ranks):
  return jax.vmap(
      lambda c, r: jnp.searchsorted(c, r, side="right").astype(jnp.int32)
  )(cum, ranks)


def kernel(x, p):
  n = x.size
  xf = x.reshape(-1)

  q = p.astype(jnp.float32) * (jnp.float32(n) - 1.0)
  low = jnp.floor(q)
  frac = q - low
  k_lo = low.astype(jnp.int32)
  ranks = jnp.stack([k_lo[0], k_lo[0] + 1, k_lo[1], k_lo[1] + 1])
  ranks = jnp.minimum(ranks, jnp.int32(n - 1))

  dummy = jnp.zeros((L,), jnp.int32)
  passes = _get_hist_passes()

  h1 = passes[0](xf, dummy).sum(axis=0)
  c1 = jnp.cumsum(h1)
  b1 = jnp.searchsorted(c1, ranks, side="right").astype(jnp.int32)
  below1 = jnp.where(b1 > 0, c1[jnp.maximum(b1 - 1, 0)], 0)
  ranks2 = ranks - below1
  pfx1 = b1

  h2 = passes[1](xf, jnp.repeat(pfx1, L)).sum(axis=0).reshape(4, 1024)
  c2 = jnp.cumsum(h2, axis=1)
  b2 = _searchsorted_rows(c2, ranks2)
  below2 = jnp.where(
      b2 > 0,
      jnp.take_along_axis(c2, jnp.maximum(b2 - 1, 0)[:, None], axis=1)[:, 0],
      0,
  )
  ranks3 = ranks2 - below2
  pfx2 = pfx1 * 1024 + b2

  h3 = passes[2](xf, jnp.repeat(pfx2, L)).sum(axis=0).reshape(4, 1024)
  c3 = jnp.cumsum(h3, axis=1)
  b3 = _searchsorted_rows(c3, ranks3)

  ku = (pfx2.astype(jnp.uint32) << 10) | b3.astype(jnp.uint32)
  half = jnp.uint32(0x80000000)
  ubits = jnp.where(ku >= half, ku - half, jnp.uint32(0) - ku)
  vals = lax.bitcast_convert_type(ubits, jnp.float32)

  q0 = vals[0] * (1.0 - frac[0]) + vals[1] * frac[0]
  q1 = vals[2] * (1.0 - frac[1]) + vals[3] * frac[1]
  return jnp.stack([q0, q1])


# speedup vs baseline: 25.4078x; 1.2126x over previous
import functools

import jax
import jax.numpy as jnp
from jax import lax
from jax.experimental import pallas as pl
from jax.experimental.pallas import tpu as pltpu
from jax.experimental.pallas import tpu_sc as plsc

NC = 2
NS = 16
NW = NC * NS
L = 16

N_TOTAL = 4096 * 4096
E_PER_W = N_TOTAL // NW
CHUNK = 8192
NCHUNKS = E_PER_W // CHUNK

PASS_CFG = (
    (20, 4096, None, 1),
    (10, 1024, 20, 4),
    (0, 1024, 10, 4),
)

def _biased_key(v):
  b = plsc.bitcast(v, jnp.int32)
  int_min = jnp.int32(-2147483648)
  neg = int_min - b
  key = jnp.where(b < 0, neg, b)
  return lax.bitwise_xor(key, int_min)


def _make_hist_pass(mesh, digit_shift, nbins, prefix_shift, nsearch):
  scratch = [
      pltpu.VMEM((CHUNK,), jnp.float32),
      pltpu.VMEM((CHUNK,), jnp.float32),
      pltpu.VMEM((nsearch * L,), jnp.int32),
      pltpu.VMEM((nsearch * nbins,), jnp.int32),
      pltpu.SemaphoreType.DMA,
      pltpu.SemaphoreType.DMA,
  ]
  for _ in range(nsearch):
    scratch.append(pltpu.VMEM((nbins * L,), jnp.int32))

  @functools.partial(
      pl.kernel,
      out_type=jax.ShapeDtypeStruct((NW, nsearch * nbins), jnp.int32),
      mesh=mesh,
      scratch_types=scratch,
      compiler_params=pltpu.CompilerParams(needs_layout_passes=False),
  )
  def hist_pass(x_hbm, params_hbm, out_hbm, bufa, bufb, pbuf, outbuf, sema,
                semb, *hists):
    wid = lax.axis_index("s") * NC + lax.axis_index("c")
    lane = lax.iota(jnp.int32, L)
    ones = jnp.ones((L,), jnp.int32)
    zeros16 = jnp.zeros((L,), jnp.int32)

    def zero_body(i, carry):
      for u in range(8):
        for h in hists:
          h[pl.ds(i * 8 * L + u * L, L)] = zeros16
      return carry

    lax.fori_loop(0, nbins // 8, zero_body, 0)

    pltpu.sync_copy(params_hbm, pbuf)
    pvs = [pbuf[pl.ds(s * L, L)] for s in range(nsearch)]

    base = wid * E_PER_W

    def consume(buf):
      def vec_body(i, inner):
        for u in range(4):
          v = buf[pl.ds(i * 4 * L + u * L, L)]
          ku = _biased_key(v)
          dig = lax.shift_right_logical(ku, digit_shift)
          if prefix_shift is None:
            idx = dig * L + lane
            plsc.addupdate_scatter(hists[0], [idx], ones)
          else:
            dig = lax.bitwise_and(dig, jnp.int32(nbins - 1))
            idx = dig * L + lane
            pfx = lax.shift_right_logical(ku, prefix_shift)
            for s in range(nsearch):
              plsc.addupdate_scatter(hists[s], [idx], ones, mask=pfx == pvs[s])
        return inner

      lax.fori_loop(0, CHUNK // L // 4, vec_body, 0)

    def chunk_slice(c):
      return x_hbm.at[pl.ds(base + c * CHUNK, CHUNK)]

    pltpu.async_copy(chunk_slice(0), bufa, sema)
    pltpu.async_copy(chunk_slice(1), bufb, semb)

    def chunk_body(g, carry):
      ca = 2 * g
      pltpu.make_async_copy(chunk_slice(ca), bufa, sema).wait()
      consume(bufa)

      @pl.when(ca + 2 < NCHUNKS)
      def _():
        pltpu.async_copy(chunk_slice(ca + 2), bufa, sema)

      pltpu.make_async_copy(chunk_slice(ca + 1), bufb, semb).wait()
      consume(bufb)

      @pl.when(ca + 3 < NCHUNKS)
      def _():
        pltpu.async_copy(chunk_slice(ca + 3), bufb, semb)

      return carry

    lax.fori_loop(0, NCHUNKS // 2, chunk_body, 0)

    for s in range(nsearch):

      def red_body(g, carry, s=s):
        bin_base = (g * L + lane) * L
        acc = jnp.zeros((L,), jnp.int32)
        for l in range(L):
          acc = acc + plsc.load_gather(hists[s], [bin_base + l])
        outbuf[pl.ds(s * nbins + g * L, L)] = acc
        return carry

      lax.fori_loop(0, nbins // L, red_body, 0)

    pltpu.sync_copy(outbuf, out_hbm.at[wid])

  return hist_pass


@functools.cache
def _get_hist_passes():
  mesh = plsc.VectorSubcoreMesh(
      core_axis_name="c", subcore_axis_name="s", num_cores=NC, num_subcores=NS
  )
  return tuple(_make_hist_pass(mesh, *cfg) for cfg in PASS_CFG)


def _searchsorted_rows(cum, ranks):
  return jax.vmap(
      lambda c, r: jnp.searchsorted(c, r, side="right").astype(jnp.int32)
  )(cum, ranks)


def kernel(x, p):
  n = x.size
  xf = x.reshape(-1)

  q = p.astype(jnp.float32) * (jnp.float32(n) - 1.0)
  low = jnp.floor(q)
  frac = q - low
  k_lo = low.astype(jnp.int32)
  ranks = jnp.stack([k_lo[0], k_lo[0] + 1, k_lo[1], k_lo[1] + 1])
  ranks = jnp.minimum(ranks, jnp.int32(n - 1))

  dummy = jnp.zeros((L,), jnp.int32)
  passes = _get_hist_passes()

  h1 = passes[0](xf, dummy).sum(axis=0)
  c1 = jnp.cumsum(h1)
  b1 = jnp.searchsorted(c1, ranks, side="right").astype(jnp.int32)
  below1 = jnp.where(b1 > 0, c1[jnp.maximum(b1 - 1, 0)], 0)
  ranks2 = ranks - below1
  pfx1 = b1

  h2 = passes[1](xf, jnp.repeat(pfx1, L)).sum(axis=0).reshape(4, 1024)
  c2 = jnp.cumsum(h2, axis=1)
  b2 = _searchsorted_rows(c2, ranks2)
  below2 = jnp.where(
      b2 > 0,
      jnp.take_along_axis(c2, jnp.maximum(b2 - 1, 0)[:, None], axis=1)[:, 0],
      0,
  )
  ranks3 = ranks2 - below2
  pfx2 = pfx1 * 1024 + b2

  h3 = passes[2](xf, jnp.repeat(pfx2, L)).sum(axis=0).reshape(4, 1024)
  c3 = jnp.cumsum(h3, axis=1)
  b3 = _searchsorted_rows(c3, ranks3)

  ku = (pfx2.astype(jnp.uint32) << 10) | b3.astype(jnp.uint32)
  half = jnp.uint32(0x80000000)
  ubits = jnp.where(ku >= half, ku - half, jnp.uint32(0) - ku)
  vals = lax.bitcast_convert_type(ubits, jnp.float32)

  q0 = vals[0] * (1.0 - frac[0]) + vals[1] * frac[0]
  q1 = vals[2] * (1.0 - frac[1]) + vals[3] * frac[1]
  return jnp.stack([q0, q1])


# speedup vs baseline: 67.2730x; 2.6477x over previous
import functools

import jax
import jax.numpy as jnp
from jax import lax
from jax.experimental import pallas as pl
from jax.experimental.pallas import tpu as pltpu
from jax.experimental.pallas import tpu_sc as plsc

NC = 2
NS = 16
NW = NC * NS
L = 16

N_TOTAL = 4096 * 4096
E_PER_W = N_TOTAL // NW
CHUNK = 8192
NCHUNKS = E_PER_W // CHUNK

PASS_CFG = (
    (20, 4096, None, 1),
    (10, 1024, 20, 4),
    (0, 1024, 10, 4),
)

def _biased_key(v):
  b = plsc.bitcast(v, jnp.int32)
  int_min = jnp.int32(-2147483648)
  neg = int_min - b
  key = jnp.where(b < 0, neg, b)
  return lax.bitwise_xor(key, int_min)


def _make_hist_pass(mesh, digit_shift, nbins, prefix_shift, nsearch):
  scratch = [
      pltpu.VMEM((CHUNK,), jnp.float32),
      pltpu.VMEM((CHUNK,), jnp.float32),
      pltpu.VMEM((nsearch * L,), jnp.int32),
      pltpu.VMEM((nsearch * nbins,), jnp.int32),
      pltpu.SemaphoreType.DMA,
      pltpu.SemaphoreType.DMA,
      pltpu.VMEM((nsearch * nbins * L,), jnp.int32),
  ]

  @functools.partial(
      pl.kernel,
      out_type=jax.ShapeDtypeStruct((NW, nsearch * nbins), jnp.int32),
      mesh=mesh,
      scratch_types=scratch,
      compiler_params=pltpu.CompilerParams(needs_layout_passes=False),
  )
  def hist_pass(x_hbm, params_hbm, out_hbm, bufa, bufb, pbuf, outbuf, sema,
                semb, hist):
    wid = lax.axis_index("s") * NC + lax.axis_index("c")
    lane = lax.iota(jnp.int32, L)
    ones = jnp.ones((L,), jnp.int32)
    zeros16 = jnp.zeros((L,), jnp.int32)

    @plsc.parallel_loop(0, nsearch * nbins, unroll=8)
    def _(i):
      hist[pl.ds(i * L, L)] = zeros16

    pltpu.sync_copy(params_hbm, pbuf)
    pvs = [pbuf[pl.ds(s * L, L)] for s in range(nsearch)]

    base = wid * E_PER_W

    def consume(buf):
      @plsc.parallel_loop(0, CHUNK // L, unroll=4)
      def _(i):
        v = buf[pl.ds(i * L, L)]
        ku = _biased_key(v)
        dig = lax.shift_right_logical(ku, digit_shift)
        if prefix_shift is None:
          idx = dig * L + lane
          plsc.addupdate_scatter(hist, [idx], ones)
        else:
          dig = lax.bitwise_and(dig, jnp.int32(nbins - 1))
          pfx = lax.shift_right_logical(ku, prefix_shift)
          masks = [pfx == pvs[s] for s in range(nsearch)]
          slot = jnp.full((L,), nsearch - 1, jnp.int32)
          any_m = masks[nsearch - 1]
          for s in range(nsearch - 2, -1, -1):
            slot = jnp.where(masks[s], jnp.int32(s), slot)
            any_m = jnp.logical_or(any_m, masks[s])
          idx = (slot * nbins + dig) * L + lane
          plsc.addupdate_scatter(hist, [idx], ones, mask=any_m)

    def chunk_slice(c):
      return x_hbm.at[pl.ds(base + c * CHUNK, CHUNK)]

    pltpu.async_copy(chunk_slice(0), bufa, sema)
    pltpu.async_copy(chunk_slice(1), bufb, semb)

    def chunk_body(g, carry):
      ca = 2 * g
      pltpu.make_async_copy(chunk_slice(ca), bufa, sema).wait()
      consume(bufa)

      @pl.when(ca + 2 < NCHUNKS)
      def _():
        pltpu.async_copy(chunk_slice(ca + 2), bufa, sema)

      pltpu.make_async_copy(chunk_slice(ca + 1), bufb, semb).wait()
      consume(bufb)

      @pl.when(ca + 3 < NCHUNKS)
      def _():
        pltpu.async_copy(chunk_slice(ca + 3), bufb, semb)

      return carry

    lax.fori_loop(0, NCHUNKS // 2, chunk_body, 0)

    for s in range(nsearch):

      @plsc.parallel_loop(0, nbins // L, unroll=2)
      def _(g, s=s):
        bin_base = (s * nbins + g * L + lane) * L
        acc = jnp.zeros((L,), jnp.int32)
        for l in range(L):
          acc = acc + plsc.load_gather(hist, [bin_base + l])
        outbuf[pl.ds(s * nbins + g * L, L)] = acc

    pltpu.sync_copy(outbuf, out_hbm.at[wid])

  return hist_pass


@functools.cache
def _get_hist_passes():
  mesh = plsc.VectorSubcoreMesh(
      core_axis_name="c", subcore_axis_name="s", num_cores=NC, num_subcores=NS
  )
  return tuple(_make_hist_pass(mesh, *cfg) for cfg in PASS_CFG)


def _dedup_rows(h_all, pfx):
  first = jnp.argmax(pfx[:, None] == pfx[None, :], axis=1)
  return h_all[first]


def _searchsorted_rows(cum, ranks):
  return jax.vmap(
      lambda c, r: jnp.searchsorted(c, r, side="right").astype(jnp.int32)
  )(cum, ranks)


def kernel(x, p):
  n = x.size
  xf = x.reshape(-1)

  q = p.astype(jnp.float32) * (jnp.float32(n) - 1.0)
  low = jnp.floor(q)
  frac = q - low
  k_lo = low.astype(jnp.int32)
  ranks = jnp.stack([k_lo[0], k_lo[0] + 1, k_lo[1], k_lo[1] + 1])
  ranks = jnp.minimum(ranks, jnp.int32(n - 1))

  dummy = jnp.zeros((L,), jnp.int32)
  passes = _get_hist_passes()

  h1 = passes[0](xf, dummy).sum(axis=0)
  c1 = jnp.cumsum(h1)
  b1 = jnp.searchsorted(c1, ranks, side="right").astype(jnp.int32)
  below1 = jnp.where(b1 > 0, c1[jnp.maximum(b1 - 1, 0)], 0)
  ranks2 = ranks - below1
  pfx1 = b1

  h2 = _dedup_rows(
      passes[1](xf, jnp.repeat(pfx1, L)).sum(axis=0).reshape(4, 1024), pfx1
  )
  c2 = jnp.cumsum(h2, axis=1)
  b2 = _searchsorted_rows(c2, ranks2)
  below2 = jnp.where(
      b2 > 0,
      jnp.take_along_axis(c2, jnp.maximum(b2 - 1, 0)[:, None], axis=1)[:, 0],
      0,
  )
  ranks3 = ranks2 - below2
  pfx2 = pfx1 * 1024 + b2

  h3 = _dedup_rows(
      passes[2](xf, jnp.repeat(pfx2, L)).sum(axis=0).reshape(4, 1024), pfx2
  )
  c3 = jnp.cumsum(h3, axis=1)
  b3 = _searchsorted_rows(c3, ranks3)

  ku = (pfx2.astype(jnp.uint32) << 10) | b3.astype(jnp.uint32)
  half = jnp.uint32(0x80000000)
  ubits = jnp.where(ku >= half, ku - half, jnp.uint32(0) - ku)
  vals = lax.bitcast_convert_type(ubits, jnp.float32)

  q0 = vals[0] * (1.0 - frac[0]) + vals[1] * frac[0]
  q1 = vals[2] * (1.0 - frac[1]) + vals[3] * frac[1]
  return jnp.stack([q0, q1])


# speedup vs baseline: 69.8209x; 1.0379x over previous
import functools

import jax
import jax.numpy as jnp
from jax import lax
from jax.experimental import pallas as pl
from jax.experimental.pallas import tpu as pltpu
from jax.experimental.pallas import tpu_sc as plsc

NC = 2
NS = 16
NW = NC * NS
L = 16

N_TOTAL = 4096 * 4096
E_PER_W = N_TOTAL // NW
CHUNK = 8192
NCHUNKS = E_PER_W // CHUNK

PASS_CFG = (
    (20, 4096, None, 1),
    (10, 1024, 20, 4),
    (0, 1024, 10, 4),
)

def _biased_key(v):
  b = plsc.bitcast(v, jnp.int32)
  int_min = jnp.int32(-2147483648)
  neg = int_min - b
  key = jnp.where(b < 0, neg, b)
  return lax.bitwise_xor(key, int_min)


def _make_hist_pass(mesh, digit_shift, nbins, prefix_shift, nsearch):
  scratch = [
      pltpu.VMEM((CHUNK,), jnp.float32),
      pltpu.VMEM((CHUNK,), jnp.float32),
      pltpu.VMEM((nsearch * L,), jnp.int32),
      pltpu.VMEM((nsearch * nbins,), jnp.int32),
      pltpu.SemaphoreType.DMA,
      pltpu.SemaphoreType.DMA,
  ]
  for _ in range(nsearch):
    scratch.append(pltpu.VMEM((nbins * L,), jnp.int32))

  @functools.partial(
      pl.kernel,
      out_type=jax.ShapeDtypeStruct((NW, nsearch * nbins), jnp.int32),
      mesh=mesh,
      scratch_types=scratch,
      compiler_params=pltpu.CompilerParams(needs_layout_passes=False),
  )
  def hist_pass(x_hbm, params_hbm, out_hbm, bufa, bufb, pbuf, outbuf, sema,
                semb, *hists):
    wid = lax.axis_index("s") * NC + lax.axis_index("c")
    lane = lax.iota(jnp.int32, L)
    ones = jnp.ones((L,), jnp.int32)
    zeros16 = jnp.zeros((L,), jnp.int32)

    @plsc.parallel_loop(0, nbins, unroll=8)
    def _(i):
      for h in hists:
        h[pl.ds(i * L, L)] = zeros16

    pltpu.sync_copy(params_hbm, pbuf)
    pvs = [pbuf[pl.ds(s * L, L)] for s in range(nsearch)]

    base = wid * E_PER_W

    def consume(buf):
      @plsc.parallel_loop(0, CHUNK // L, unroll=8)
      def _(i):
        v = buf[pl.ds(i * L, L)]
        ku = _biased_key(v)
        dig = lax.shift_right_logical(ku, digit_shift)
        if prefix_shift is None:
          idx = dig * L + lane
          plsc.addupdate_scatter(hists[0], [idx], ones)
        else:
          dig = lax.bitwise_and(dig, jnp.int32(nbins - 1))
          idx = dig * L + lane
          pfx = lax.shift_right_logical(ku, prefix_shift)
          for s in range(nsearch):
            plsc.addupdate_scatter(hists[s], [idx], ones, mask=pfx == pvs[s])

    def chunk_slice(c):
      return x_hbm.at[pl.ds(base + c * CHUNK, CHUNK)]

    pltpu.async_copy(chunk_slice(0), bufa, sema)
    pltpu.async_copy(chunk_slice(1), bufb, semb)

    def chunk_body(g, carry):
      ca = 2 * g
      pltpu.make_async_copy(chunk_slice(ca), bufa, sema).wait()
      consume(bufa)

      @pl.when(ca + 2 < NCHUNKS)
      def _():
        pltpu.async_copy(chunk_slice(ca + 2), bufa, sema)

      pltpu.make_async_copy(chunk_slice(ca + 1), bufb, semb).wait()
      consume(bufb)

      @pl.when(ca + 3 < NCHUNKS)
      def _():
        pltpu.async_copy(chunk_slice(ca + 3), bufb, semb)

      return carry

    lax.fori_loop(0, NCHUNKS // 2, chunk_body, 0)

    for s in range(nsearch):

      @plsc.parallel_loop(0, nbins // L, unroll=2)
      def _(g, s=s):
        bin_base = (g * L + lane) * L
        acc = jnp.zeros((L,), jnp.int32)
        for l in range(L):
          acc = acc + plsc.load_gather(hists[s], [bin_base + l])
        outbuf[pl.ds(s * nbins + g * L, L)] = acc

    pltpu.sync_copy(outbuf, out_hbm.at[wid])

  return hist_pass


@functools.cache
def _get_hist_passes():
  mesh = plsc.VectorSubcoreMesh(
      core_axis_name="c", subcore_axis_name="s", num_cores=NC, num_subcores=NS
  )
  return tuple(_make_hist_pass(mesh, *cfg) for cfg in PASS_CFG)


def _searchsorted_rows(cum, ranks):
  return jax.vmap(
      lambda c, r: jnp.searchsorted(c, r, side="right").astype(jnp.int32)
  )(cum, ranks)


def kernel(x, p):
  n = x.size
  xf = x.reshape(-1)

  q = p.astype(jnp.float32) * (jnp.float32(n) - 1.0)
  low = jnp.floor(q)
  frac = q - low
  k_lo = low.astype(jnp.int32)
  ranks = jnp.stack([k_lo[0], k_lo[0] + 1, k_lo[1], k_lo[1] + 1])
  ranks = jnp.minimum(ranks, jnp.int32(n - 1))

  dummy = jnp.zeros((L,), jnp.int32)
  passes = _get_hist_passes()

  h1 = passes[0](xf, dummy).sum(axis=0)
  c1 = jnp.cumsum(h1)
  b1 = jnp.searchsorted(c1, ranks, side="right").astype(jnp.int32)
  below1 = jnp.where(b1 > 0, c1[jnp.maximum(b1 - 1, 0)], 0)
  ranks2 = ranks - below1
  pfx1 = b1

  h2 = passes[1](xf, jnp.repeat(pfx1, L)).sum(axis=0).reshape(4, 1024)
  c2 = jnp.cumsum(h2, axis=1)
  b2 = _searchsorted_rows(c2, ranks2)
  below2 = jnp.where(
      b2 > 0,
      jnp.take_along_axis(c2, jnp.maximum(b2 - 1, 0)[:, None], axis=1)[:, 0],
      0,
  )
  ranks3 = ranks2 - below2
  pfx2 = pfx1 * 1024 + b2

  h3 = passes[2](xf, jnp.repeat(pfx2, L)).sum(axis=0).reshape(4, 1024)
  c3 = jnp.cumsum(h3, axis=1)
  b3 = _searchsorted_rows(c3, ranks3)

  ku = (pfx2.astype(jnp.uint32) << 10) | b3.astype(jnp.uint32)
  half = jnp.uint32(0x80000000)
  ubits = jnp.where(ku >= half, ku - half, jnp.uint32(0) - ku)
  vals = lax.bitcast_convert_type(ubits, jnp.float32)

  q0 = vals[0] * (1.0 - frac[0]) + vals[1] * frac[0]
  q1 = vals[2] * (1.0 - frac[1]) + vals[3] * frac[1]
  return jnp.stack([q0, q1])


# speedup vs baseline: 78.0304x; 1.1176x over previous
import functools

import jax
import jax.numpy as jnp
from jax import lax
from jax.experimental import pallas as pl
from jax.experimental.pallas import tpu as pltpu
from jax.experimental.pallas import tpu_sc as plsc

NC = 2
NS = 16
NW = NC * NS
L = 16

N_TOTAL = 4096 * 4096
E_PER_W = N_TOTAL // NW
CHUNK = 8192
NCHUNKS = E_PER_W // CHUNK

PASS_CFG = (
    (20, 4096, None, 1),
    (10, 1024, 20, 4),
    (0, 1024, 10, 4),
)

def _biased_key(v):
  b = plsc.bitcast(v, jnp.int32)
  int_min = jnp.int32(-2147483648)
  neg = int_min - b
  key = jnp.where(b < 0, neg, b)
  return lax.bitwise_xor(key, int_min)


def _make_hist_pass(mesh, digit_shift, nbins, prefix_shift, nsearch):
  scratch = [
      pltpu.VMEM((CHUNK,), jnp.float32),
      pltpu.VMEM((CHUNK,), jnp.float32),
      pltpu.VMEM((nsearch * L,), jnp.int32),
      pltpu.VMEM((nsearch * nbins,), jnp.int32),
      pltpu.SemaphoreType.DMA,
      pltpu.SemaphoreType.DMA,
  ]
  for _ in range(nsearch):
    scratch.append(pltpu.VMEM((nbins * L,), jnp.int32))

  @functools.partial(
      pl.kernel,
      out_type=jax.ShapeDtypeStruct((NW, nsearch * nbins), jnp.int32),
      mesh=mesh,
      scratch_types=scratch,
      compiler_params=pltpu.CompilerParams(needs_layout_passes=False),
  )
  def hist_pass(x_hbm, params_hbm, out_hbm, bufa, bufb, pbuf, outbuf, sema,
                semb, *hists):
    wid = lax.axis_index("s") * NC + lax.axis_index("c")
    lane = lax.iota(jnp.int32, L)
    ones = jnp.ones((L,), jnp.int32)
    zeros16 = jnp.zeros((L,), jnp.int32)

    @plsc.parallel_loop(0, nbins, unroll=8)
    def _(i):
      for h in hists:
        h[pl.ds(i * L, L)] = zeros16

    pltpu.sync_copy(params_hbm, pbuf)
    pvs = [pbuf[pl.ds(s * L, L)] for s in range(nsearch)]

    base = wid * E_PER_W

    def consume(buf):
      @plsc.parallel_loop(0, CHUNK // L, unroll=4)
      def _(i):
        v = buf[pl.ds(i * L, L)]
        ku = _biased_key(v)
        dig = lax.shift_right_logical(ku, digit_shift)
        if prefix_shift is None:
          idx = dig * L + lane
          plsc.addupdate_scatter(hists[0], [idx], ones)
        else:
          dig = lax.bitwise_and(dig, jnp.int32(nbins - 1))
          idx = dig * L + lane
          pfx = lax.shift_right_logical(ku, prefix_shift)
          for s in range(nsearch):
            plsc.addupdate_scatter(hists[s], [idx], ones, mask=pfx == pvs[s])

    def chunk_slice(c):
      return x_hbm.at[pl.ds(base + c * CHUNK, CHUNK)]

    pltpu.async_copy(chunk_slice(0), bufa, sema)
    pltpu.async_copy(chunk_slice(1), bufb, semb)

    def chunk_body(g, carry):
      ca = 2 * g
      pltpu.make_async_copy(chunk_slice(ca), bufa, sema).wait()
      consume(bufa)

      @pl.when(ca + 2 < NCHUNKS)
      def _():
        pltpu.async_copy(chunk_slice(ca + 2), bufa, sema)

      pltpu.make_async_copy(chunk_slice(ca + 1), bufb, semb).wait()
      consume(bufb)

      @pl.when(ca + 3 < NCHUNKS)
      def _():
        pltpu.async_copy(chunk_slice(ca + 3), bufb, semb)

      return carry

    lax.fori_loop(0, NCHUNKS // 2, chunk_body, 0)

    for s in range(nsearch):

      @plsc.parallel_loop(0, nbins // L, unroll=2)
      def _(g, s=s):
        bin_base = (g * L + lane) * L
        acc = jnp.zeros((L,), jnp.int32)
        for l in range(L):
          acc = acc + plsc.load_gather(hists[s], [bin_base + l])
        outbuf[pl.ds(s * nbins + g * L, L)] = acc

    pltpu.sync_copy(outbuf, out_hbm.at[wid])

  return hist_pass


@functools.cache
def _get_hist_passes():
  mesh = plsc.VectorSubcoreMesh(
      core_axis_name="c", subcore_axis_name="s", num_cores=NC, num_subcores=NS
  )
  return tuple(_make_hist_pass(mesh, *cfg) for cfg in PASS_CFG)


def _searchsorted_rows(cum, ranks):
  return jax.vmap(
      lambda c, r: jnp.searchsorted(c, r, side="right").astype(jnp.int32)
  )(cum, ranks)


def kernel(x, p):
  n = x.size
  xf = x.reshape(-1)

  q = p.astype(jnp.float32) * (jnp.float32(n) - 1.0)
  low = jnp.floor(q)
  frac = q - low
  k_lo = low.astype(jnp.int32)
  ranks = jnp.stack([k_lo[0], k_lo[0] + 1, k_lo[1], k_lo[1] + 1])
  ranks = jnp.minimum(ranks, jnp.int32(n - 1))

  dummy = jnp.zeros((L,), jnp.int32)
  passes = _get_hist_passes()

  h1 = passes[0](xf, dummy).sum(axis=0)
  c1 = jnp.cumsum(h1)
  b1 = jnp.searchsorted(c1, ranks, side="right").astype(jnp.int32)
  below1 = jnp.where(b1 > 0, c1[jnp.maximum(b1 - 1, 0)], 0)
  ranks2 = ranks - below1
  pfx1 = b1

  h2 = passes[1](xf, jnp.repeat(pfx1, L)).sum(axis=0).reshape(4, 1024)
  c2 = jnp.cumsum(h2, axis=1)
  b2 = _searchsorted_rows(c2, ranks2)
  below2 = jnp.where(
      b2 > 0,
      jnp.take_along_axis(c2, jnp.maximum(b2 - 1, 0)[:, None], axis=1)[:, 0],
      0,
  )
  ranks3 = ranks2 - below2
  pfx2 = pfx1 * 1024 + b2

  h3 = passes[2](xf, jnp.repeat(pfx2, L)).sum(axis=0).reshape(4, 1024)
  c3 = jnp.cumsum(h3, axis=1)
  b3 = _searchsorted_rows(c3, ranks3)

  ku = (pfx2.astype(jnp.uint32) << 10) | b3.astype(jnp.uint32)
  half = jnp.uint32(0x80000000)
  ubits = jnp.where(ku >= half, ku - half, jnp.uint32(0) - ku)
  vals = lax.bitcast_convert_type(ubits, jnp.float32)

  q0 = vals[0] * (1.0 - frac[0]) + vals[1] * frac[0]
  q1 = vals[2] * (1.0 - frac[1]) + vals[3] * frac[1]
  return jnp.stack([q0, q1])


# speedup vs baseline: 79.7323x; 1.0218x over previous
import functools

import jax
import jax.numpy as jnp
from jax import lax
from jax.experimental import pallas as pl
from jax.experimental.pallas import tpu as pltpu
from jax.experimental.pallas import tpu_sc as plsc

NC = 2
NS = 16
NW = NC * NS
L = 16

N_TOTAL = 4096 * 4096
E_PER_W = N_TOTAL // NW
CHUNK = 16384
NCHUNKS = E_PER_W // CHUNK

PASS_CFG = (
    (20, 4096, None, 1),
    (10, 1024, 20, 4),
    (0, 1024, 10, 4),
)

def _biased_key(v):
  b = plsc.bitcast(v, jnp.int32)
  int_min = jnp.int32(-2147483648)
  neg = int_min - b
  key = jnp.where(b < 0, neg, b)
  return lax.bitwise_xor(key, int_min)


def _make_hist_pass(mesh, digit_shift, nbins, prefix_shift, nsearch):
  scratch = [
      pltpu.VMEM((CHUNK,), jnp.float32),
      pltpu.VMEM((CHUNK,), jnp.float32),
      pltpu.VMEM((nsearch * L,), jnp.int32),
      pltpu.VMEM((nsearch * nbins,), jnp.int32),
      pltpu.SemaphoreType.DMA,
      pltpu.SemaphoreType.DMA,
  ]
  for _ in range(nsearch):
    scratch.append(pltpu.VMEM((nbins * L,), jnp.int32))

  @functools.partial(
      pl.kernel,
      out_type=jax.ShapeDtypeStruct((NW, nsearch * nbins), jnp.int32),
      mesh=mesh,
      scratch_types=scratch,
      compiler_params=pltpu.CompilerParams(needs_layout_passes=False),
  )
  def hist_pass(x_hbm, params_hbm, out_hbm, bufa, bufb, pbuf, outbuf, sema,
                semb, *hists):
    wid = lax.axis_index("s") * NC + lax.axis_index("c")
    lane = lax.iota(jnp.int32, L)
    ones = jnp.ones((L,), jnp.int32)
    zeros16 = jnp.zeros((L,), jnp.int32)

    @plsc.parallel_loop(0, nbins, unroll=8)
    def _(i):
      for h in hists:
        h[pl.ds(i * L, L)] = zeros16

    pltpu.sync_copy(params_hbm, pbuf)
    pvs = [pbuf[pl.ds(s * L, L)] for s in range(nsearch)]

    base = wid * E_PER_W

    def consume(buf):
      @plsc.parallel_loop(0, CHUNK // L, unroll=4)
      def _(i):
        v = buf[pl.ds(i * L, L)]
        ku = _biased_key(v)
        dig = lax.shift_right_logical(ku, digit_shift)
        if prefix_shift is None:
          idx = dig * L + lane
          plsc.addupdate_scatter(hists[0], [idx], ones)
        else:
          dig = lax.bitwise_and(dig, jnp.int32(nbins - 1))
          idx = dig * L + lane
          pfx = lax.shift_right_logical(ku, prefix_shift)
          for s in range(nsearch):
            plsc.addupdate_scatter(hists[s], [idx], ones, mask=pfx == pvs[s])

    def chunk_slice(c):
      return x_hbm.at[pl.ds(base + c * CHUNK, CHUNK)]

    pltpu.async_copy(chunk_slice(0), bufa, sema)
    pltpu.async_copy(chunk_slice(1), bufb, semb)

    def chunk_body(g, carry):
      ca = 2 * g
      pltpu.make_async_copy(chunk_slice(ca), bufa, sema).wait()
      consume(bufa)

      @pl.when(ca + 2 < NCHUNKS)
      def _():
        pltpu.async_copy(chunk_slice(ca + 2), bufa, sema)

      pltpu.make_async_copy(chunk_slice(ca + 1), bufb, semb).wait()
      consume(bufb)

      @pl.when(ca + 3 < NCHUNKS)
      def _():
        pltpu.async_copy(chunk_slice(ca + 3), bufb, semb)

      return carry

    lax.fori_loop(0, NCHUNKS // 2, chunk_body, 0)

    for s in range(nsearch):

      @plsc.parallel_loop(0, nbins // L, unroll=2)
      def _(g, s=s):
        bin_base = (g * L + lane) * L
        acc = jnp.zeros((L,), jnp.int32)
        for l in range(L):
          acc = acc + plsc.load_gather(hists[s], [bin_base + l])
        outbuf[pl.ds(s * nbins + g * L, L)] = acc

    pltpu.sync_copy(outbuf, out_hbm.at[wid])

  return hist_pass


@functools.cache
def _get_hist_passes():
  mesh = plsc.VectorSubcoreMesh(
      core_axis_name="c", subcore_axis_name="s", num_cores=NC, num_subcores=NS
  )
  return tuple(_make_hist_pass(mesh, *cfg) for cfg in PASS_CFG)


def _searchsorted_rows(cum, ranks):
  return jax.vmap(
      lambda c, r: jnp.searchsorted(c, r, side="right").astype(jnp.int32)
  )(cum, ranks)


def kernel(x, p):
  n = x.size
  xf = x.reshape(-1)

  q = p.astype(jnp.float32) * (jnp.float32(n) - 1.0)
  low = jnp.floor(q)
  frac = q - low
  k_lo = low.astype(jnp.int32)
  ranks = jnp.stack([k_lo[0], k_lo[0] + 1, k_lo[1], k_lo[1] + 1])
  ranks = jnp.minimum(ranks, jnp.int32(n - 1))

  dummy = jnp.zeros((L,), jnp.int32)
  passes = _get_hist_passes()

  h1 = passes[0](xf, dummy).sum(axis=0)
  c1 = jnp.cumsum(h1)
  b1 = jnp.searchsorted(c1, ranks, side="right").astype(jnp.int32)
  below1 = jnp.where(b1 > 0, c1[jnp.maximum(b1 - 1, 0)], 0)
  ranks2 = ranks - below1
  pfx1 = b1

  h2 = passes[1](xf, jnp.repeat(pfx1, L)).sum(axis=0).reshape(4, 1024)
  c2 = jnp.cumsum(h2, axis=1)
  b2 = _searchsorted_rows(c2, ranks2)
  below2 = jnp.where(
      b2 > 0,
      jnp.take_along_axis(c2, jnp.maximum(b2 - 1, 0)[:, None], axis=1)[:, 0],
      0,
  )
  ranks3 = ranks2 - below2
  pfx2 = pfx1 * 1024 + b2

  h3 = passes[2](xf, jnp.repeat(pfx2, L)).sum(axis=0).reshape(4, 1024)
  c3 = jnp.cumsum(h3, axis=1)
  b3 = _searchsorted_rows(c3, ranks3)

  ku = (pfx2.astype(jnp.uint32) << 10) | b3.astype(jnp.uint32)
  half = jnp.uint32(0x80000000)
  ubits = jnp.where(ku >= half, ku - half, jnp.uint32(0) - ku)
  vals = lax.bitcast_convert_type(ubits, jnp.float32)

  q0 = vals[0] * (1.0 - frac[0]) + vals[1] * frac[0]
  q1 = vals[2] * (1.0 - frac[1]) + vals[3] * frac[1]
  return jnp.stack([q0, q1])
